# baseline (speedup 1.0000x reference)
import functools
import math

import jax
import jax.numpy as jnp
from jax import lax
from jax.experimental import pallas as pl
from jax.experimental.pallas import tpu as pltpu
from jax.experimental.pallas import tpu_sc as plsc

_D = 512
_SCALE = math.sqrt(_D)
_LANES = 16
_NC = 2
_NS = 16


@functools.lru_cache(maxsize=None)
def _build(B):
    NW = _NC * _NS
    b_per_w = B // NW
    C = 64
    n_chunks = b_per_w // C

    mesh = plsc.VectorSubcoreMesh(core_axis_name="c", subcore_axis_name="s")

    @functools.partial(
        pl.kernel,
        mesh=mesh,
        out_type=jax.ShapeDtypeStruct((B, _D), jnp.float32),
        scratch_types=[
            pltpu.VMEM((C,), jnp.int32),
            pltpu.VMEM((C, _D), jnp.float32),
            pltpu.SemaphoreType.DMA,
        ],
    )
    def emb(tok_hbm, table_hbm, out_hbm, idx_v, buf, gsem):
        wid = lax.axis_index("s") * _NC + lax.axis_index("c")
        base = wid * b_per_w

        def chunk_body(i, carry):
            row0 = base + i * C
            pltpu.sync_copy(tok_hbm.at[pl.ds(row0, C)], idx_v)
            pltpu.async_copy(table_hbm.at[idx_v], buf, gsem).wait()

            def row_body(r, c2):
                for j in range(_D // _LANES):
                    sl = pl.ds(j * _LANES, _LANES)
                    buf[r, sl] = buf[r, sl] * _SCALE
                return c2

            lax.fori_loop(0, C, row_body, 0)
            pltpu.sync_copy(buf, out_hbm.at[pl.ds(row0, C)])
            return carry

        lax.fori_loop(0, n_chunks, chunk_body, 0)

    return emb


def kernel(tokens, table):
    B = tokens.shape[0] * tokens.shape[1]
    flat = tokens.reshape(B)
    out = _build(B)(flat, table)
    return out.reshape(tokens.shape + (_D,))


# speedup vs baseline: 1.1956x; 1.1956x over previous
import functools
import math

import jax
import jax.numpy as jnp
from jax import lax
from jax.experimental import pallas as pl
from jax.experimental.pallas import tpu as pltpu
from jax.experimental.pallas import tpu_sc as plsc

_D = 512
_SCALE = math.sqrt(_D)
_LANES = 16
_NC = 2
_NS = 16


@functools.lru_cache(maxsize=None)
def _build(B):
    NW = _NC * _NS
    b_per_w = B // NW
    C = 80
    n_chunks = b_per_w // C
    n_pairs = n_chunks // 2

    mesh = plsc.VectorSubcoreMesh(core_axis_name="c", subcore_axis_name="s")

    @functools.partial(
        pl.kernel,
        mesh=mesh,
        out_type=jax.ShapeDtypeStruct((B, _D), jnp.float32),
        scratch_types=[
            pltpu.VMEM((C,), jnp.int32),
            pltpu.VMEM((C,), jnp.int32),
            pltpu.VMEM((C, _D), jnp.float32),
            pltpu.VMEM((C, _D), jnp.float32),
            pltpu.SemaphoreType.DMA,
            pltpu.SemaphoreType.DMA,
        ],
    )
    def emb(tok_hbm, table_hbm, out_hbm, idx0, idx1, buf0, buf1, gsem0, gsem1):
        wid = lax.axis_index("s") * _NC + lax.axis_index("c")
        base = wid * b_per_w
        idx = (idx0, idx1)
        buf = (buf0, buf1)
        gsem = (gsem0, gsem1)

        def prefetch(b, i):
            row0 = base + i * C
            pltpu.sync_copy(tok_hbm.at[pl.ds(row0, C)], idx[b])
            pltpu.async_copy(table_hbm.at[idx[b]], buf[b], gsem[b])

        def process(b, i):
            row0 = base + i * C
            pltpu.make_async_copy(table_hbm.at[idx[b]], buf[b], gsem[b]).wait()

            def row_body(r, c2):
                for j in range(_D // _LANES):
                    sl = pl.ds(j * _LANES, _LANES)
                    buf[b][r, sl] = buf[b][r, sl] * _SCALE
                return c2

            lax.fori_loop(0, C, row_body, 0)
            pltpu.sync_copy(buf[b], out_hbm.at[pl.ds(row0, C)])

        prefetch(0, 0)
        prefetch(1, 1)

        def pair_body(g, carry):
            i0 = 2 * g
            process(0, i0)
            prefetch(0, i0 + 2)
            process(1, i0 + 1)
            prefetch(1, i0 + 3)
            return carry

        lax.fori_loop(0, n_pairs - 1, pair_body, 0)
        i0 = 2 * (n_pairs - 1)
        process(0, i0)
        process(1, i0 + 1)

    return emb


def kernel(tokens, table):
    B = tokens.shape[0] * tokens.shape[1]
    flat = tokens.reshape(B)
    out = _build(B)(flat, table)
    return out.reshape(tokens.shape + (_D,))


# speedup vs baseline: 1.1981x; 1.0021x over previous
import functools
import math

import jax
import jax.numpy as jnp
from jax import lax
from jax.experimental import pallas as pl
from jax.experimental.pallas import tpu as pltpu
from jax.experimental.pallas import tpu_sc as plsc

_D = 512
_SCALE = math.sqrt(_D)
_LANES = 16
_NC = 2
_NS = 16


@functools.lru_cache(maxsize=None)
def _build(B):
    NW = _NC * _NS
    b_per_w = B // NW
    C = 64
    n = b_per_w // C

    mesh = plsc.VectorSubcoreMesh(core_axis_name="c", subcore_axis_name="s")

    @functools.partial(
        pl.kernel,
        mesh=mesh,
        out_type=jax.ShapeDtypeStruct((B, _D), jnp.float32),
        scratch_types=[
            pltpu.VMEM((C,), jnp.int32),
            pltpu.VMEM((C,), jnp.int32),
            pltpu.VMEM((C,), jnp.int32),
            pltpu.VMEM((C, _D), jnp.float32),
            pltpu.VMEM((C, _D), jnp.float32),
            pltpu.VMEM((C, _D), jnp.float32),
            pltpu.SemaphoreType.DMA,
            pltpu.SemaphoreType.DMA,
            pltpu.SemaphoreType.DMA,
            pltpu.SemaphoreType.DMA,
            pltpu.SemaphoreType.DMA,
            pltpu.SemaphoreType.DMA,
        ],
    )
    def emb(tok_hbm, table_hbm, out_hbm, i0, i1, i2, b0, b1, b2,
            g0, g1, g2, s0, s1, s2):
        wid = lax.axis_index("s") * _NC + lax.axis_index("c")
        base = wid * b_per_w
        idx = (i0, i1, i2)
        buf = (b0, b1, b2)
        gsem = (g0, g1, g2)
        ssem = (s0, s1, s2)

        def gather(b, i):
            row0 = base + i * C
            pltpu.sync_copy(tok_hbm.at[pl.ds(row0, C)], idx[b])
            pltpu.async_copy(table_hbm.at[idx[b]], buf[b], gsem[b])

        def wait_gather(b):
            pltpu.make_async_copy(table_hbm.at[idx[b]], buf[b], gsem[b]).wait()

        def scale(b):
            def row_body(r, c2):
                for j in range(_D // _LANES):
                    sl = pl.ds(j * _LANES, _LANES)
                    buf[b][r, sl] = buf[b][r, sl] * _SCALE
                return c2

            lax.fori_loop(0, C, row_body, 0)

        def scatter(b, i):
            row0 = base + i * C
            pltpu.async_copy(buf[b], out_hbm.at[pl.ds(row0, C)], ssem[b])

        def wait_scatter(b, i):
            row0 = base + i * C
            pltpu.make_async_copy(
                buf[b], out_hbm.at[pl.ds(row0, C)], ssem[b]).wait()

        def step(b, i, prefetch):
            b2_ = (b + 2) % 3
            wait_gather(b)
            scale(b)
            scatter(b, i)
            wait_scatter(b2_, i - 1)
            if prefetch:
                gather(b2_, i + 2)

        gather(0, 0)
        gather(1, 1)
        wait_gather(0); scale(0); scatter(0, 0); gather(2, 2)
        wait_gather(1); scale(1); scatter(1, 1)
        wait_scatter(0, 0); gather(0, 3)
        wait_gather(2); scale(2); scatter(2, 2)
        wait_scatter(1, 1); gather(1, 4)

        def group_body(g, carry):
            i = 3 * g
            step(0, i, True)
            step(1, i + 1, True)
            step(2, i + 2, True)
            return carry

        lax.fori_loop(1, (n - 4) // 3, group_body, 0)

        step(0, n - 4, True)
        step(1, n - 3, True)
        step(2, n - 2, False)
        step(0, n - 1, False)
        wait_scatter(0, n - 1)

    return emb


def kernel(tokens, table):
    B = tokens.shape[0] * tokens.shape[1]
    flat = tokens.reshape(B)
    out = _build(B)(flat, table)
    return out.reshape(tokens.shape + (_D,))


# speedup vs baseline: 3.7586x; 3.1372x over previous
import functools
import math

import jax
import jax.numpy as jnp
from jax import lax
from jax.experimental import pallas as pl
from jax.experimental.pallas import tpu as pltpu
from jax.experimental.pallas import tpu_sc as plsc

_D = 512
_SCALE = math.sqrt(_D)
_LANES = 16
_NC = 2
_NS = 16


@functools.lru_cache(maxsize=None)
def _build(T, W):
    NW = _NC * _NS
    rows_w = T // NW
    C = 64
    halves = rows_w // C
    n = W * halves

    mesh = plsc.VectorSubcoreMesh(core_axis_name="c", subcore_axis_name="s")

    @functools.partial(
        pl.kernel,
        mesh=mesh,
        out_type=jax.ShapeDtypeStruct((W, T, _D), jnp.float32),
        scratch_types=[
            pltpu.VMEM((W, rows_w), jnp.int32),
            pltpu.VMEM((C, _D), jnp.float32),
            pltpu.VMEM((C, _D), jnp.float32),
            pltpu.VMEM((C, _D), jnp.float32),
            pltpu.SemaphoreType.DMA,
            pltpu.SemaphoreType.DMA,
            pltpu.SemaphoreType.DMA,
            pltpu.SemaphoreType.DMA,
            pltpu.SemaphoreType.DMA,
            pltpu.SemaphoreType.DMA,
        ],
    )
    def emb(tokt_hbm, table_hbm, out_hbm, idx_v, b0, b1, b2,
            g0, g1, g2, s0, s1, s2):
        wid = lax.axis_index("s") * _NC + lax.axis_index("c")
        row0 = wid * rows_w
        buf = (b0, b1, b2)
        gsem = (g0, g1, g2)
        ssem = (s0, s1, s2)

        pltpu.sync_copy(tokt_hbm.at[:, pl.ds(row0, rows_w)], idx_v)

        def idx_ref(i):
            return idx_v.at[i // halves, pl.ds((i % halves) * C, C)]

        def gather(b, i):
            pltpu.async_copy(table_hbm.at[idx_ref(i)], buf[b], gsem[b])

        def wait_gather(b, i):
            pltpu.make_async_copy(
                table_hbm.at[idx_ref(i)], buf[b], gsem[b]).wait()

        def scale(b):
            @plsc.parallel_loop(0, C, step=1, unroll=1)
            def row_body(r):
                for j in range(_D // _LANES):
                    sl = pl.ds(j * _LANES, _LANES)
                    buf[b][r, sl] = buf[b][r, sl] * _SCALE

        def out_slice(i):
            return out_hbm.at[i // halves,
                              pl.ds(row0 + (i % halves) * C, C), :]

        def scatter(b, i):
            pltpu.async_copy(buf[b], out_slice(i), ssem[b])

        def wait_scatter(b, i):
            pltpu.make_async_copy(buf[b], out_slice(i), ssem[b]).wait()

        def step(b, i, prefetch):
            b2_ = (b + 2) % 3
            wait_gather(b, i)
            scale(b)
            scatter(b, i)
            wait_scatter(b2_, i - 1)
            if prefetch:
                gather(b2_, i + 2)

        gather(0, 0)
        gather(1, 1)
        wait_gather(0, 0); scale(0); scatter(0, 0); gather(2, 2)
        wait_gather(1, 1); scale(1); scatter(1, 1)
        wait_scatter(0, 0); gather(0, 3)
        wait_gather(2, 2); scale(2); scatter(2, 2)
        wait_scatter(1, 1); gather(1, 4)

        def group_body(g, carry):
            i = 3 * g
            step(0, i, True)
            step(1, i + 1, True)
            step(2, i + 2, True)
            return carry

        lax.fori_loop(1, (n - 4) // 3, group_body, 0)

        step(0, n - 4, True)
        step(1, n - 3, True)
        step(2, n - 2, False)
        step(0, n - 1, False)
        wait_scatter(0, n - 1)

    return emb


def kernel(tokens, table):
    T, W = tokens.shape
    out = _build(T, W)(tokens.T, table)
    return jnp.transpose(out, (1, 0, 2))
